# baseline (speedup 1.0000x reference)
import functools

import jax
import jax.numpy as jnp
from jax import lax
from jax.experimental import pallas as pl
from jax.experimental.pallas import tpu as pltpu
from jax.experimental.pallas import tpu_sc as plsc

B = 4096
D = 32
ATT = 20
MAXLEN = 200
ALPHA = 0.5
IVOCAB = 1000
VOCAB = 1000000
TAIL = VOCAB - (VOCAB // 128) * 128
TAIL_START = VOCAB - TAIL
LAST_BLOCK = (VOCAB // 128) - 1
NC = 2
NS = 16
L = 16
NW = NC * NS
CB = B // NW
NG = CB // L
NSLOT = 3

_mesh = plsc.VectorSubcoreMesh(core_axis_name="c", subcore_axis_name="s")


@functools.partial(
    pl.kernel,
    out_type=jax.ShapeDtypeStruct((B * 2,), jnp.float32),
    mesh=_mesh,
    compiler_params=pltpu.CompilerParams(needs_layout_passes=False),
    scratch_types=[
        pltpu.VMEM((CB,), jnp.int32),
        pltpu.VMEM((CB,), jnp.int32),
        pltpu.VMEM((CB,), jnp.int32),
        pltpu.VMEM((CB,), jnp.int32),
        pltpu.VMEM((CB,), jnp.int32),
        pltpu.VMEM((L, MAXLEN), jnp.int32),
        pltpu.VMEM((L, MAXLEN), jnp.int32),
        pltpu.VMEM((D, IVOCAB), jnp.float32),
        pltpu.VMEM((D, IVOCAB), jnp.float32),
        pltpu.VMEM((D, TAIL), jnp.float32),
        pltpu.VMEM((D, TAIL), jnp.float32),
        pltpu.VMEM((2 * L,), jnp.float32),
        pltpu.VMEM((NSLOT, D, 128), jnp.float32),
        pltpu.VMEM((NSLOT, D, 128), jnp.float32),
        pltpu.VMEM((NSLOT, D, 128), jnp.float32),
        pltpu.VMEM((L * D,), jnp.float32),
        pltpu.VMEM((L * D,), jnp.float32),
        pltpu.VMEM((L * D,), jnp.float32),
        pltpu.VMEM((L * D,), jnp.float32),
        pltpu.VMEM((CB * 2,), jnp.float32),
        pltpu.SemaphoreType.DMA,
        pltpu.SemaphoreType.DMA,
        pltpu.SemaphoreType.DMA,
        pltpu.SemaphoreType.DMA,
    ],
)
def _imrec_sc(uidx_h, ipos_h, ineg_h, tpos_h, tneg_h, iseq_h, mseq_h,
              ftab_h, ttab_h, utab_h, itab_h, utail_h, itail_h, w_h, out_h,
              uidx_v, ipos_v, ineg_v, tpos_v, tneg_v, iseq_c, mseq_c,
              ftab_v, ttab_v, utail_v, itail_v, w_v,
              ublk, pblk, nblk, urow_f, iprow_f, inrow_f, fi_f,
              out_v, sem_tab, sem_s0, sem_s1, sem_s2):
    wid = lax.axis_index("s") * NC + lax.axis_index("c")
    base = wid * CB

    pltpu.async_copy(ftab_h, ftab_v, sem_tab)
    pltpu.async_copy(ttab_h, ttab_v, sem_tab)
    pltpu.async_copy(utail_h, utail_v, sem_tab)
    pltpu.async_copy(itail_h, itail_v, sem_tab)

    pltpu.sync_copy(uidx_h.at[pl.ds(base, CB)], uidx_v)
    pltpu.sync_copy(ipos_h.at[pl.ds(base, CB)], ipos_v)
    pltpu.sync_copy(ineg_h.at[pl.ds(base, CB)], ineg_v)
    pltpu.sync_copy(tpos_h.at[pl.ds(base, CB)], tpos_v)
    pltpu.sync_copy(tneg_h.at[pl.ds(base, CB)], tneg_v)
    pltpu.sync_copy(w_h, w_v)

    pltpu.make_async_copy(ftab_h, ftab_v, sem_tab).wait()
    pltpu.make_async_copy(ttab_h, ttab_v, sem_tab).wait()
    pltpu.make_async_copy(utail_h, utail_v, sem_tab).wait()
    pltpu.make_async_copy(itail_h, itail_v, sem_tab).wait()

    iota16 = lax.broadcasted_iota(jnp.int32, (L,), 0)
    iota16b = iota16 + L
    zeros = jnp.zeros((L,), jnp.float32)
    sems = (sem_s0, sem_s1, sem_s2)

    def fire(slot, su, sp, sn):
        sem = sems[slot]
        pltpu.async_copy(utab_h.at[:, pl.ds(pl.multiple_of(su, 128), 128)],
                         ublk.at[slot], sem)
        pltpu.async_copy(itab_h.at[:, pl.ds(pl.multiple_of(sp, 128), 128)],
                         pblk.at[slot], sem)
        pltpu.async_copy(itab_h.at[:, pl.ds(pl.multiple_of(sn, 128), 128)],
                         nblk.at[slot], sem)

    def drain(slot):
        sem = sems[slot]
        pltpu.make_async_copy(utab_h.at[:, pl.ds(0, 128)], ublk.at[slot],
                              sem).wait()
        pltpu.make_async_copy(itab_h.at[:, pl.ds(0, 128)], pblk.at[slot],
                              sem).wait()
        pltpu.make_async_copy(itab_h.at[:, pl.ds(0, 128)], nblk.at[slot],
                              sem).wait()

    def extract(blk, slot, tail_v, col, tcol, tsel, dst_f, dst_off):
        sidx = jnp.full((L,), slot, jnp.int32)
        lo = plsc.load_gather(blk, [sidx, iota16, col])
        hi = plsc.load_gather(blk, [sidx, iota16b, col])
        tlo = plsc.load_gather(tail_v, [iota16, tcol])
        thi = plsc.load_gather(tail_v, [iota16b, tcol])
        lo = jnp.where(tsel, tlo, lo)
        hi = jnp.where(tsel, thi, hi)
        plsc.store_scatter(dst_f, [dst_off + iota16], lo)
        plsc.store_scatter(dst_f, [dst_off + iota16b], hi)

    @pl.loop(0, NG, unroll=1)
    def _chunk(g):
        off = g * L
        uu = uidx_v[pl.ds(off, L)]
        pp = ipos_v[pl.ds(off, L)]
        nn = ineg_v[pl.ds(off, L)]
        su_vec = jnp.minimum(uu // 128, LAST_BLOCK) * 128
        sp_vec = jnp.minimum(pp // 128, LAST_BLOCK) * 128
        sn_vec = jnp.minimum(nn // 128, LAST_BLOCK) * 128
        cu_vec = jnp.minimum(uu - su_vec, 127)
        cp_vec = jnp.minimum(pp - sp_vec, 127)
        cn_vec = jnp.minimum(nn - sn_vec, 127)
        tu_vec = jnp.maximum(uu - TAIL_START, 0)
        tp_vec = jnp.maximum(pp - TAIL_START, 0)
        tn_vec = jnp.maximum(nn - TAIL_START, 0)

        fire(0, su_vec[0], sp_vec[0], sn_vec[0])
        fire(1, su_vec[1], sp_vec[1], sn_vec[1])
        fire(2, su_vec[2], sp_vec[2], sn_vec[2])

        pltpu.sync_copy(iseq_h.at[pl.ds(base + off, L), :], iseq_c)
        pltpu.sync_copy(mseq_h.at[pl.ds(base + off, L), :], mseq_c)

        rows_d0 = iota16 * D
        idxs = []
        wms = []
        for i in range(ATT):
            col = jnp.full((L,), MAXLEN - 1 - i, jnp.int32)
            idx = plsc.load_gather(iseq_c, [iota16, col])
            item = plsc.load_gather(mseq_c, [iota16, col])
            wi = plsc.load_gather(w_v, [jnp.full((L,), i, jnp.int32)])
            wms.append(jnp.where(item == 0, 0.0, wi))
            idxs.append(idx)
        for d in range(D):
            dcol = jnp.full((L,), d, jnp.int32)
            fparts = [zeros, zeros, zeros, zeros]
            for i in range(ATT):
                v = plsc.load_gather(ftab_v, [dcol, idxs[i]])
                fparts[i % 4] = fparts[i % 4] + wms[i] * v
            fi = (fparts[0] + fparts[1]) + (fparts[2] + fparts[3])
            plsc.store_scatter(fi_f, [rows_d0 + d], fi)

        for j in range(L):
            slot = j % NSLOT
            drain(slot)
            dst_off = jnp.full((L,), j * D, jnp.int32)
            usel = jnp.full((L,), uu[j], jnp.int32) >= TAIL_START
            psel = jnp.full((L,), pp[j], jnp.int32) >= TAIL_START
            nsel = jnp.full((L,), nn[j], jnp.int32) >= TAIL_START
            extract(ublk, slot, utail_v,
                    jnp.full((L,), cu_vec[j], jnp.int32),
                    jnp.full((L,), tu_vec[j], jnp.int32), usel, urow_f,
                    dst_off)
            extract(pblk, slot, itail_v,
                    jnp.full((L,), cp_vec[j], jnp.int32),
                    jnp.full((L,), tp_vec[j], jnp.int32), psel, iprow_f,
                    dst_off)
            extract(nblk, slot, itail_v,
                    jnp.full((L,), cn_vec[j], jnp.int32),
                    jnp.full((L,), tn_vec[j], jnp.int32), nsel, inrow_f,
                    dst_off)
            if j + NSLOT < L:
                fire(slot, su_vec[j + NSLOT], sp_vec[j + NSLOT],
                     sn_vec[j + NSLOT])

        lanes = iota16 + off
        tposv = plsc.load_gather(tpos_v, [lanes])
        tnegv = plsc.load_gather(tneg_v, [lanes])
        lp = [zeros, zeros]
        ln = [zeros, zeros]
        sp = [zeros, zeros]
        sn = [zeros, zeros]
        for d in range(D):
            dcol = jnp.full((L,), d, jnp.int32)
            pos = rows_d0 + d
            u = plsc.load_gather(urow_f, [pos])
            ip = plsc.load_gather(iprow_f, [pos])
            inn = plsc.load_gather(inrow_f, [pos])
            fi = plsc.load_gather(fi_f, [pos])
            tp = plsc.load_gather(ttab_v, [dcol, tposv])
            tn = plsc.load_gather(ttab_v, [dcol, tnegv])
            k = d % 2
            lp[k] = lp[k] + u * ip
            ln[k] = ln[k] + u * inn
            sp[k] = sp[k] + fi * tp
            sn[k] = sn[k] + fi * tn
        score_p = ALPHA * (lp[0] + lp[1]) + (1.0 - ALPHA) * (sp[0] + sp[1])
        score_n = ALPHA * (ln[0] + ln[1]) + (1.0 - ALPHA) * (sn[0] + sn[1])
        prob_p = 1.0 / (1.0 + jnp.exp(-score_p))
        prob_n = 1.0 / (1.0 + jnp.exp(-score_n))
        lanes2 = lanes * 2
        plsc.store_scatter(out_v, [lanes2], prob_p)
        plsc.store_scatter(out_v, [lanes2 + 1], prob_n)

    pltpu.sync_copy(out_v, out_h.at[pl.ds(base * 2, CB * 2)])


def kernel(user_id, item_id_seq, type_seq, intention_seq, next_item_seq,
           next_item_neg_seq, target_item_pos, target_intention_pos,
           target_item_neg, target_intention_neg,
           user_table, item_table, from_intention_table, to_intention_table,
           timestep_w):
    del type_seq, next_item_seq, next_item_neg_seq
    uidx = user_id.reshape(B)
    ipos = target_item_pos.reshape(B)
    ineg = target_item_neg.reshape(B)
    tpos = target_intention_pos.reshape(B)
    tneg = target_intention_neg.reshape(B)
    utab_t = user_table.T
    itab_t = item_table.T
    utail_t = user_table[TAIL_START:, :].T
    itail_t = item_table[TAIL_START:, :].T
    ftab_t = from_intention_table.T
    ttab_t = to_intention_table.T
    w_pad = jnp.pad(timestep_w, (0, 2 * L - ATT))
    flat = _imrec_sc(uidx, ipos, ineg, tpos, tneg, intention_seq, item_id_seq,
                     ftab_t, ttab_t, utab_t, itab_t, utail_t, itail_t, w_pad)
    return flat.reshape(B, 2)


# speedup vs baseline: 1.0985x; 1.0985x over previous
import functools

import jax
import jax.numpy as jnp
from jax import lax
from jax.experimental import pallas as pl
from jax.experimental.pallas import tpu as pltpu
from jax.experimental.pallas import tpu_sc as plsc

B = 4096
D = 32
ATT = 20
MAXLEN = 200
ALPHA = 0.5
IVOCAB = 1000
VOCAB = 1000000
TAIL = VOCAB - (VOCAB // 128) * 128
TAIL_START = VOCAB - TAIL
LAST_BLOCK = (VOCAB // 128) - 1
NC = 2
NS = 16
L = 16
NW = NC * NS
CB = B // NW
NG = CB // L
NSLOT = 3

_mesh = plsc.VectorSubcoreMesh(core_axis_name="c", subcore_axis_name="s")


@functools.partial(
    pl.kernel,
    out_type=jax.ShapeDtypeStruct((B * 2,), jnp.float32),
    mesh=_mesh,
    compiler_params=pltpu.CompilerParams(needs_layout_passes=False),
    scratch_types=[
        pltpu.VMEM((CB,), jnp.int32),
        pltpu.VMEM((CB,), jnp.int32),
        pltpu.VMEM((CB,), jnp.int32),
        pltpu.VMEM((CB,), jnp.int32),
        pltpu.VMEM((CB,), jnp.int32),
        pltpu.VMEM((CB * ATT,), jnp.int32),
        pltpu.VMEM((CB * ATT,), jnp.int32),
        pltpu.VMEM((D, IVOCAB), jnp.float32),
        pltpu.VMEM((D, IVOCAB), jnp.float32),
        pltpu.VMEM((D, TAIL), jnp.float32),
        pltpu.VMEM((D, TAIL), jnp.float32),
        pltpu.VMEM((2 * L,), jnp.float32),
        pltpu.VMEM((NSLOT, D, 128), jnp.float32),
        pltpu.VMEM((NSLOT, D, 128), jnp.float32),
        pltpu.VMEM((NSLOT, D, 128), jnp.float32),
        pltpu.VMEM((L * D,), jnp.float32),
        pltpu.VMEM((L * D,), jnp.float32),
        pltpu.VMEM((L * D,), jnp.float32),
        pltpu.VMEM((L * D,), jnp.float32),
        pltpu.VMEM((CB * 2,), jnp.float32),
        pltpu.SemaphoreType.DMA,
        pltpu.SemaphoreType.DMA,
        pltpu.SemaphoreType.DMA,
        pltpu.SemaphoreType.DMA,
    ],
)
def _imrec_sc(uidx_h, ipos_h, ineg_h, tpos_h, tneg_h, iseq_h, mseq_h,
              ftab_h, ttab_h, utab_h, itab_h, utail_h, itail_h, w_h, out_h,
              uidx_v, ipos_v, ineg_v, tpos_v, tneg_v, iseq_v, mseq_v,
              ftab_v, ttab_v, utail_v, itail_v, w_v,
              ublk, pblk, nblk, urow_f, iprow_f, inrow_f, fi_f,
              out_v, sem_tab, sem_s0, sem_s1, sem_s2):
    wid = lax.axis_index("s") * NC + lax.axis_index("c")
    base = wid * CB

    pltpu.async_copy(ftab_h, ftab_v, sem_tab)
    pltpu.async_copy(ttab_h, ttab_v, sem_tab)
    pltpu.async_copy(utail_h, utail_v, sem_tab)
    pltpu.async_copy(itail_h, itail_v, sem_tab)

    pltpu.sync_copy(uidx_h.at[pl.ds(base, CB)], uidx_v)
    pltpu.sync_copy(ipos_h.at[pl.ds(base, CB)], ipos_v)
    pltpu.sync_copy(ineg_h.at[pl.ds(base, CB)], ineg_v)
    pltpu.sync_copy(tpos_h.at[pl.ds(base, CB)], tpos_v)
    pltpu.sync_copy(tneg_h.at[pl.ds(base, CB)], tneg_v)
    pltpu.sync_copy(iseq_h.at[pl.ds(base * ATT, CB * ATT)], iseq_v)
    pltpu.sync_copy(mseq_h.at[pl.ds(base * ATT, CB * ATT)], mseq_v)
    pltpu.sync_copy(w_h, w_v)

    pltpu.make_async_copy(ftab_h, ftab_v, sem_tab).wait()
    pltpu.make_async_copy(ttab_h, ttab_v, sem_tab).wait()
    pltpu.make_async_copy(utail_h, utail_v, sem_tab).wait()
    pltpu.make_async_copy(itail_h, itail_v, sem_tab).wait()

    iota16 = lax.broadcasted_iota(jnp.int32, (L,), 0)
    iota16b = iota16 + L
    zeros = jnp.zeros((L,), jnp.float32)
    sems = (sem_s0, sem_s1, sem_s2)

    def fire(slot, su, sp, sn):
        sem = sems[slot]
        pltpu.async_copy(utab_h.at[:, pl.ds(pl.multiple_of(su, 128), 128)],
                         ublk.at[slot], sem)
        pltpu.async_copy(itab_h.at[:, pl.ds(pl.multiple_of(sp, 128), 128)],
                         pblk.at[slot], sem)
        pltpu.async_copy(itab_h.at[:, pl.ds(pl.multiple_of(sn, 128), 128)],
                         nblk.at[slot], sem)

    def drain(slot):
        sem = sems[slot]
        pltpu.make_async_copy(utab_h.at[:, pl.ds(0, 128)], ublk.at[slot],
                              sem).wait()
        pltpu.make_async_copy(itab_h.at[:, pl.ds(0, 128)], pblk.at[slot],
                              sem).wait()
        pltpu.make_async_copy(itab_h.at[:, pl.ds(0, 128)], nblk.at[slot],
                              sem).wait()

    def extract(blk, slot, tail_v, col, tcol, tsel, dst_f, dst_off):
        sidx = jnp.full((L,), slot, jnp.int32)
        lo = plsc.load_gather(blk, [sidx, iota16, col])
        hi = plsc.load_gather(blk, [sidx, iota16b, col])
        tlo = plsc.load_gather(tail_v, [iota16, tcol])
        thi = plsc.load_gather(tail_v, [iota16b, tcol])
        lo = jnp.where(tsel, tlo, lo)
        hi = jnp.where(tsel, thi, hi)
        plsc.store_scatter(dst_f, [dst_off + iota16], lo)
        plsc.store_scatter(dst_f, [dst_off + iota16b], hi)

    @pl.loop(0, NG, unroll=1)
    def _chunk(g):
        off = g * L
        uu = uidx_v[pl.ds(off, L)]
        pp = ipos_v[pl.ds(off, L)]
        nn = ineg_v[pl.ds(off, L)]
        su_vec = jnp.minimum(uu // 128, LAST_BLOCK) * 128
        sp_vec = jnp.minimum(pp // 128, LAST_BLOCK) * 128
        sn_vec = jnp.minimum(nn // 128, LAST_BLOCK) * 128
        cu_vec = jnp.minimum(uu - su_vec, 127)
        cp_vec = jnp.minimum(pp - sp_vec, 127)
        cn_vec = jnp.minimum(nn - sn_vec, 127)
        tu_vec = jnp.maximum(uu - TAIL_START, 0)
        tp_vec = jnp.maximum(pp - TAIL_START, 0)
        tn_vec = jnp.maximum(nn - TAIL_START, 0)

        fire(0, su_vec[0], sp_vec[0], sn_vec[0])
        fire(1, su_vec[1], sp_vec[1], sn_vec[1])
        fire(2, su_vec[2], sp_vec[2], sn_vec[2])

        lanes = iota16 + off
        lanes_seq = lanes * ATT
        rows_d0 = iota16 * D
        idxs = []
        wms = []
        for i in range(ATT):
            pos = lanes_seq + (ATT - 1 - i)
            idx = plsc.load_gather(iseq_v, [pos])
            item = plsc.load_gather(mseq_v, [pos])
            wi = plsc.load_gather(w_v, [jnp.full((L,), i, jnp.int32)])
            wms.append(jnp.where(item == 0, 0.0, wi))
            idxs.append(idx)
        for d in range(D):
            dcol = jnp.full((L,), d, jnp.int32)
            fparts = [zeros, zeros, zeros, zeros]
            for i in range(ATT):
                v = plsc.load_gather(ftab_v, [dcol, idxs[i]])
                fparts[i % 4] = fparts[i % 4] + wms[i] * v
            fi = (fparts[0] + fparts[1]) + (fparts[2] + fparts[3])
            plsc.store_scatter(fi_f, [rows_d0 + d], fi)

        for j in range(L):
            slot = j % NSLOT
            drain(slot)
            dst_off = jnp.full((L,), j * D, jnp.int32)
            usel = jnp.full((L,), uu[j], jnp.int32) >= TAIL_START
            psel = jnp.full((L,), pp[j], jnp.int32) >= TAIL_START
            nsel = jnp.full((L,), nn[j], jnp.int32) >= TAIL_START
            extract(ublk, slot, utail_v,
                    jnp.full((L,), cu_vec[j], jnp.int32),
                    jnp.full((L,), tu_vec[j], jnp.int32), usel, urow_f,
                    dst_off)
            extract(pblk, slot, itail_v,
                    jnp.full((L,), cp_vec[j], jnp.int32),
                    jnp.full((L,), tp_vec[j], jnp.int32), psel, iprow_f,
                    dst_off)
            extract(nblk, slot, itail_v,
                    jnp.full((L,), cn_vec[j], jnp.int32),
                    jnp.full((L,), tn_vec[j], jnp.int32), nsel, inrow_f,
                    dst_off)
            if j + NSLOT < L:
                fire(slot, su_vec[j + NSLOT], sp_vec[j + NSLOT],
                     sn_vec[j + NSLOT])

        tposv = plsc.load_gather(tpos_v, [lanes])
        tnegv = plsc.load_gather(tneg_v, [lanes])
        lp = [zeros, zeros]
        ln = [zeros, zeros]
        sp = [zeros, zeros]
        sn = [zeros, zeros]
        for d in range(D):
            dcol = jnp.full((L,), d, jnp.int32)
            pos = rows_d0 + d
            u = plsc.load_gather(urow_f, [pos])
            ip = plsc.load_gather(iprow_f, [pos])
            inn = plsc.load_gather(inrow_f, [pos])
            fi = plsc.load_gather(fi_f, [pos])
            tp = plsc.load_gather(ttab_v, [dcol, tposv])
            tn = plsc.load_gather(ttab_v, [dcol, tnegv])
            k = d % 2
            lp[k] = lp[k] + u * ip
            ln[k] = ln[k] + u * inn
            sp[k] = sp[k] + fi * tp
            sn[k] = sn[k] + fi * tn
        score_p = ALPHA * (lp[0] + lp[1]) + (1.0 - ALPHA) * (sp[0] + sp[1])
        score_n = ALPHA * (ln[0] + ln[1]) + (1.0 - ALPHA) * (sn[0] + sn[1])
        prob_p = 1.0 / (1.0 + jnp.exp(-score_p))
        prob_n = 1.0 / (1.0 + jnp.exp(-score_n))
        lanes2 = lanes * 2
        plsc.store_scatter(out_v, [lanes2], prob_p)
        plsc.store_scatter(out_v, [lanes2 + 1], prob_n)

    pltpu.sync_copy(out_v, out_h.at[pl.ds(base * 2, CB * 2)])


def kernel(user_id, item_id_seq, type_seq, intention_seq, next_item_seq,
           next_item_neg_seq, target_item_pos, target_intention_pos,
           target_item_neg, target_intention_neg,
           user_table, item_table, from_intention_table, to_intention_table,
           timestep_w):
    del type_seq, next_item_seq, next_item_neg_seq
    uidx = user_id.reshape(B)
    ipos = target_item_pos.reshape(B)
    ineg = target_item_neg.reshape(B)
    tpos = target_intention_pos.reshape(B)
    tneg = target_intention_neg.reshape(B)
    utab_t = user_table.T
    itab_t = item_table.T
    utail_t = user_table[TAIL_START:, :].T
    itail_t = item_table[TAIL_START:, :].T
    ftab_t = from_intention_table.T
    ttab_t = to_intention_table.T
    w_pad = jnp.pad(timestep_w, (0, 2 * L - ATT))
    iseq = intention_seq[:, -ATT:].reshape(B * ATT)
    mseq = item_id_seq[:, -ATT:].reshape(B * ATT)
    flat = _imrec_sc(uidx, ipos, ineg, tpos, tneg, iseq, mseq,
                     ftab_t, ttab_t, utab_t, itab_t, utail_t, itail_t, w_pad)
    return flat.reshape(B, 2)


# speedup vs baseline: 1.1000x; 1.0014x over previous
import functools

import jax
import jax.numpy as jnp
from jax import lax
from jax.experimental import pallas as pl
from jax.experimental.pallas import tpu as pltpu
from jax.experimental.pallas import tpu_sc as plsc

B = 4096
D = 32
ATT = 20
ALPHA = 0.5
IVOCAB = 1000
VOCAB = 1000000
TAIL = VOCAB - (VOCAB // 128) * 128
TAIL_START = VOCAB - TAIL
LAST_BLOCK = (VOCAB // 128) - 1
NC = 2
NS = 16
L = 16
NW = NC * NS
CB = B // NW
NG = CB // L
NSLOT = 3

_mesh = plsc.VectorSubcoreMesh(core_axis_name="c", subcore_axis_name="s")


@functools.partial(
    pl.kernel,
    out_type=jax.ShapeDtypeStruct((B * 2,), jnp.float32),
    mesh=_mesh,
    compiler_params=pltpu.CompilerParams(needs_layout_passes=False),
    scratch_types=[
        pltpu.VMEM((CB,), jnp.int32),
        pltpu.VMEM((CB,), jnp.int32),
        pltpu.VMEM((CB,), jnp.int32),
        pltpu.VMEM((CB,), jnp.int32),
        pltpu.VMEM((CB,), jnp.int32),
        pltpu.VMEM((CB * ATT,), jnp.int32),
        pltpu.VMEM((CB * ATT,), jnp.int32),
        pltpu.VMEM((D, IVOCAB), jnp.float32),
        pltpu.VMEM((D, IVOCAB), jnp.float32),
        pltpu.VMEM((D, TAIL), jnp.float32),
        pltpu.VMEM((D, TAIL), jnp.float32),
        pltpu.VMEM((2 * L,), jnp.float32),
        pltpu.VMEM((NSLOT, D, 128), jnp.float32),
        pltpu.VMEM((NSLOT, D, 128), jnp.float32),
        pltpu.VMEM((NSLOT, D, 128), jnp.float32),
        pltpu.VMEM((L * D,), jnp.float32),
        pltpu.VMEM((L * D,), jnp.float32),
        pltpu.VMEM((L * D,), jnp.float32),
        pltpu.VMEM((L * D,), jnp.float32),
        pltpu.VMEM((CB * 2,), jnp.float32),
        pltpu.SemaphoreType.DMA,
        pltpu.SemaphoreType.DMA,
        pltpu.SemaphoreType.DMA,
        pltpu.SemaphoreType.DMA,
    ],
)
def _imrec_sc(uidx_h, ipos_h, ineg_h, tpos_h, tneg_h, iseq_h, mseq_h,
              ftab_h, ttab_h, utab_h, itab_h, utail_h, itail_h, w_h, out_h,
              uidx_v, ipos_v, ineg_v, tpos_v, tneg_v, iseq_v, mseq_v,
              ftab_v, ttab_v, utail_v, itail_v, w_v,
              ublk, pblk, nblk, urow_f, iprow_f, inrow_f, fi_f,
              out_v, sem_tab, sem_s0, sem_s1, sem_s2):
    wid = lax.axis_index("s") * NC + lax.axis_index("c")
    base = wid * CB

    pltpu.async_copy(ftab_h, ftab_v, sem_tab)
    pltpu.async_copy(ttab_h, ttab_v, sem_tab)
    pltpu.async_copy(utail_h, utail_v, sem_tab)
    pltpu.async_copy(itail_h, itail_v, sem_tab)

    pltpu.sync_copy(uidx_h.at[pl.ds(base, CB)], uidx_v)
    pltpu.sync_copy(ipos_h.at[pl.ds(base, CB)], ipos_v)
    pltpu.sync_copy(ineg_h.at[pl.ds(base, CB)], ineg_v)
    pltpu.sync_copy(tpos_h.at[pl.ds(base, CB)], tpos_v)
    pltpu.sync_copy(tneg_h.at[pl.ds(base, CB)], tneg_v)
    pltpu.sync_copy(iseq_h.at[pl.ds(base * ATT, CB * ATT)], iseq_v)
    pltpu.sync_copy(mseq_h.at[pl.ds(base * ATT, CB * ATT)], mseq_v)
    pltpu.sync_copy(w_h, w_v)

    pltpu.make_async_copy(ftab_h, ftab_v, sem_tab).wait()
    pltpu.make_async_copy(ttab_h, ttab_v, sem_tab).wait()
    pltpu.make_async_copy(utail_h, utail_v, sem_tab).wait()
    pltpu.make_async_copy(itail_h, itail_v, sem_tab).wait()

    iota16 = lax.broadcasted_iota(jnp.int32, (L,), 0)
    iota16b = iota16 + L
    zeros = jnp.zeros((L,), jnp.float32)
    sems = (sem_s0, sem_s1, sem_s2)

    def fire(slot, su, sp, sn):
        sem = sems[slot]
        pltpu.async_copy(utab_h.at[:, pl.ds(pl.multiple_of(su, 128), 128)],
                         ublk.at[slot], sem)
        pltpu.async_copy(itab_h.at[:, pl.ds(pl.multiple_of(sp, 128), 128)],
                         pblk.at[slot], sem)
        pltpu.async_copy(itab_h.at[:, pl.ds(pl.multiple_of(sn, 128), 128)],
                         nblk.at[slot], sem)

    def drain(slot):
        sem = sems[slot]
        pltpu.make_async_copy(utab_h.at[:, pl.ds(0, 128)], ublk.at[slot],
                              sem).wait()
        pltpu.make_async_copy(itab_h.at[:, pl.ds(0, 128)], pblk.at[slot],
                              sem).wait()
        pltpu.make_async_copy(itab_h.at[:, pl.ds(0, 128)], nblk.at[slot],
                              sem).wait()

    def extract(blk, slot, tail_v, col, tcol, tsel, dst_f, dst_off):
        sidx = jnp.full((L,), slot, jnp.int32)
        lo = plsc.load_gather(blk, [sidx, iota16, col])
        hi = plsc.load_gather(blk, [sidx, iota16b, col])
        tlo = plsc.load_gather(tail_v, [iota16, tcol])
        thi = plsc.load_gather(tail_v, [iota16b, tcol])
        lo = jnp.where(tsel, tlo, lo)
        hi = jnp.where(tsel, thi, hi)
        plsc.store_scatter(dst_f, [dst_off + iota16], lo)
        plsc.store_scatter(dst_f, [dst_off + iota16b], hi)

    @pl.loop(0, NG, unroll=1)
    def _chunk(g):
        off = g * L
        uu = uidx_v[pl.ds(off, L)]
        pp = ipos_v[pl.ds(off, L)]
        nn = ineg_v[pl.ds(off, L)]
        su_vec = jnp.minimum(uu // 128, LAST_BLOCK) * 128
        sp_vec = jnp.minimum(pp // 128, LAST_BLOCK) * 128
        sn_vec = jnp.minimum(nn // 128, LAST_BLOCK) * 128
        cu_vec = jnp.minimum(uu - su_vec, 127)
        cp_vec = jnp.minimum(pp - sp_vec, 127)
        cn_vec = jnp.minimum(nn - sn_vec, 127)
        tu_vec = jnp.maximum(uu - TAIL_START, 0)
        tp_vec = jnp.maximum(pp - TAIL_START, 0)
        tn_vec = jnp.maximum(nn - TAIL_START, 0)

        fire(0, su_vec[0], sp_vec[0], sn_vec[0])
        fire(1, su_vec[1], sp_vec[1], sn_vec[1])
        fire(2, su_vec[2], sp_vec[2], sn_vec[2])

        lanes = iota16 + off
        lanes_seq = lanes * ATT
        rows_d0 = iota16 * D
        idxs = []
        wms = []
        for i in range(ATT):
            pos = lanes_seq + (ATT - 1 - i)
            idx = plsc.load_gather(iseq_v, [pos])
            item = plsc.load_gather(mseq_v, [pos])
            wi = plsc.load_gather(w_v, [jnp.full((L,), i, jnp.int32)])
            wms.append(jnp.where(item == 0, 0.0, wi))
            idxs.append(idx)
        for d in range(D):
            dcol = jnp.full((L,), d, jnp.int32)
            fparts = [zeros, zeros, zeros, zeros]
            for i in range(ATT):
                v = plsc.load_gather(ftab_v, [dcol, idxs[i]])
                fparts[i % 4] = fparts[i % 4] + wms[i] * v
            fi = (fparts[0] + fparts[1]) + (fparts[2] + fparts[3])
            plsc.store_scatter(fi_f, [rows_d0 + d], fi)

        for j in range(L):
            slot = j % NSLOT
            drain(slot)
            dst_off = jnp.full((L,), j * D, jnp.int32)
            usel = jnp.full((L,), uu[j], jnp.int32) >= TAIL_START
            psel = jnp.full((L,), pp[j], jnp.int32) >= TAIL_START
            nsel = jnp.full((L,), nn[j], jnp.int32) >= TAIL_START
            extract(ublk, slot, utail_v,
                    jnp.full((L,), cu_vec[j], jnp.int32),
                    jnp.full((L,), tu_vec[j], jnp.int32), usel, urow_f,
                    dst_off)
            extract(pblk, slot, itail_v,
                    jnp.full((L,), cp_vec[j], jnp.int32),
                    jnp.full((L,), tp_vec[j], jnp.int32), psel, iprow_f,
                    dst_off)
            extract(nblk, slot, itail_v,
                    jnp.full((L,), cn_vec[j], jnp.int32),
                    jnp.full((L,), tn_vec[j], jnp.int32), nsel, inrow_f,
                    dst_off)
            if j + NSLOT < L:
                fire(slot, su_vec[j + NSLOT], sp_vec[j + NSLOT],
                     sn_vec[j + NSLOT])

        tposv = plsc.load_gather(tpos_v, [lanes])
        tnegv = plsc.load_gather(tneg_v, [lanes])
        lp = [zeros, zeros]
        ln = [zeros, zeros]
        sp = [zeros, zeros]
        sn = [zeros, zeros]
        for d in range(D):
            dcol = jnp.full((L,), d, jnp.int32)
            pos = rows_d0 + d
            u = plsc.load_gather(urow_f, [pos])
            ip = plsc.load_gather(iprow_f, [pos])
            inn = plsc.load_gather(inrow_f, [pos])
            fi = plsc.load_gather(fi_f, [pos])
            tp = plsc.load_gather(ttab_v, [dcol, tposv])
            tn = plsc.load_gather(ttab_v, [dcol, tnegv])
            k = d % 2
            lp[k] = lp[k] + u * ip
            ln[k] = ln[k] + u * inn
            sp[k] = sp[k] + fi * tp
            sn[k] = sn[k] + fi * tn
        score_p = ALPHA * (lp[0] + lp[1]) + (1.0 - ALPHA) * (sp[0] + sp[1])
        score_n = ALPHA * (ln[0] + ln[1]) + (1.0 - ALPHA) * (sn[0] + sn[1])
        prob_p = 1.0 / (1.0 + jnp.exp(-score_p))
        prob_n = 1.0 / (1.0 + jnp.exp(-score_n))
        lanes2 = lanes * 2
        plsc.store_scatter(out_v, [lanes2], prob_p)
        plsc.store_scatter(out_v, [lanes2 + 1], prob_n)

    pltpu.sync_copy(out_v, out_h.at[pl.ds(base * 2, CB * 2)])


def kernel(user_id, item_id_seq, type_seq, intention_seq, next_item_seq,
           next_item_neg_seq, target_item_pos, target_intention_pos,
           target_item_neg, target_intention_neg,
           user_table, item_table, from_intention_table, to_intention_table,
           timestep_w):
    del type_seq, next_item_seq, next_item_neg_seq
    uidx = user_id.reshape(B)
    ipos = target_item_pos.reshape(B)
    ineg = target_item_neg.reshape(B)
    tpos = target_intention_pos.reshape(B)
    tneg = target_intention_neg.reshape(B)
    utab_t = user_table.T
    itab_t = item_table.T
    utail_t = user_table[TAIL_START:, :].T
    itail_t = item_table[TAIL_START:, :].T
    ftab_t = from_intention_table.T
    ttab_t = to_intention_table.T
    w_pad = jnp.pad(timestep_w, (0, 2 * L - ATT))
    iseq = intention_seq[:, -ATT:].reshape(B * ATT)
    mseq = item_id_seq[:, -ATT:].reshape(B * ATT)
    flat = _imrec_sc(uidx, ipos, ineg, tpos, tneg, iseq, mseq,
                     ftab_t, ttab_t, utab_t, itab_t, utail_t, itail_t, w_pad)
    return flat.reshape(B, 2)
